# Initial kernel scaffold; baseline (speedup 1.0000x reference)
import jax
import jax.numpy as jnp
from jax.experimental import pallas as pl


def kernel(x, edge_index, batch, W1, b1, W2, b2, Wl, bl):
    raise NotImplementedError("write your pallas kernel here")


# speedup vs baseline: 1.5478x; 1.5478x over previous
import jax
import jax.numpy as jnp
from jax import lax
from jax.experimental import pallas as pl
from jax.experimental.pallas import tpu as pltpu
from jax.experimental.pallas import tpu_sc as plsc

_N = 10000
_E = 320000
_IN_C = 128
_HID = 256
_OUT_C = 64
_G = 64

_NP = 10240
_NW = 32
_CPW = _HID // _NW

_DEPT = _E // _NW
_DK = 400
_DNCH = _DEPT // _DK
_RPW = _NP // _NW

_RK = 64
_RE = _RK * 16
_NR = 313
_EPAD = _NR * _RE
_PAIRS = _RE // 2
_PU = 8

_BR = 256
_NB = _NP // _BR


def _mesh():
    return plsc.VectorSubcoreMesh(core_axis_name="c", subcore_axis_name="s")


def _wid():
    return lax.axis_index("c") * 16 + lax.axis_index("s")


def _sc_degree(dst, zer_np):
    def body(dst_hbm, zer_hbm, deg_hbm, part, hist, dacc, dbuf, didx):
        w = _wid()
        pltpu.sync_copy(zer_hbm, hist)
        ones16 = jnp.ones((16,), jnp.float32)

        def chunk(i, carry):
            base = pl.multiple_of(w * _DEPT + i * _DK, 8)
            pltpu.sync_copy(dst_hbm.at[pl.ds(base, _DK)], didx)
            for k in range(_DK // 16):
                d16 = didx[pl.ds(k * 16, 16)]
                plsc.addupdate_scatter(hist, [d16], ones16)
            return carry

        lax.fori_loop(0, _DNCH, chunk, 0)
        for r in range(_NW):
            pltpu.sync_copy(hist.at[pl.ds(r * _RPW, _RPW)],
                            part.at[w * _NW + r])
        plsc.subcore_barrier()

        pltpu.sync_copy(part.at[w], dacc)
        for t in range(1, _NW):
            pltpu.sync_copy(part.at[t * _NW + w], dbuf)
            for k in range(_RPW // 16):
                dacc[pl.ds(k * 16, 16)] += dbuf[pl.ds(k * 16, 16)]
        r0 = pl.multiple_of(w * _RPW, 8)
        pltpu.sync_copy(dacc, deg_hbm.at[pl.ds(r0, _RPW)])

    kfn = pl.kernel(
        body,
        out_type=jax.ShapeDtypeStruct((_NP,), jnp.float32),
        mesh=_mesh(),
        scratch_types=[
            pltpu.VMEM_SHARED((_NW * _NW, _RPW), jnp.float32),
            pltpu.VMEM((_NP,), jnp.float32),
            pltpu.VMEM((_RPW,), jnp.float32),
            pltpu.VMEM((_RPW,), jnp.float32),
            pltpu.VMEM((_DK,), jnp.int32),
        ],
        compiler_params=pltpu.CompilerParams(needs_layout_passes=False, use_tc_tiling_on_sc=False),
    )
    return kfn(dst, zer_np)


def _sc_scatter(y3, srcp, dstp, zer8):
    def body(y_hbm, src_hbm, dst_hbm, zer_hbm, z_hbm,
             acc, rows, rbuf, sidx, didx, sbuf, sem):
        c = lax.axis_index("c")
        s = lax.axis_index("s")
        w = c * 16 + s
        pltpu.sync_copy(zer_hbm, acc)
        iota = lax.iota(jnp.int32, 16)
        pat = jnp.right_shift(iota, 3)
        colofs = jnp.bitwise_and(iota, 7)

        def round_(r, carry):
            e0 = r * _RE
            base = pl.multiple_of(e0 + s * _RK, 8)
            pltpu.sync_copy(src_hbm.at[pl.ds(base, _RK)], sidx)
            pltpu.async_copy(y_hbm.at[c].at[plsc.Indices(sidx)],
                             rows, sem).wait()
            for d in range(16):
                pltpu.sync_copy(rows.at[:, pl.ds(d * _CPW, _CPW)],
                                sbuf.at[d, pl.ds(s * _RK, _RK)])
            e0a = pl.multiple_of(e0, 8)
            pltpu.sync_copy(dst_hbm.at[pl.ds(e0a, _RE)], didx)
            plsc.subcore_barrier()
            pltpu.sync_copy(sbuf.at[s], rbuf)

            def pairs(p, carry2):
                for u in range(_PU):
                    rpat = (p * _PU + u) * 2 + pat
                    a16 = plsc.load_gather(didx, [rpat])
                    v16 = plsc.load_gather(rbuf, [rpat, colofs])
                    plsc.addupdate_scatter(acc, [a16, colofs], v16)
                return carry2

            lax.fori_loop(0, _PAIRS // _PU, pairs, 0)
            plsc.subcore_barrier()
            return carry

        lax.fori_loop(0, _NR, round_, 0)
        pltpu.sync_copy(acc, z_hbm.at[w])

    kfn = pl.kernel(
        body,
        out_type=jax.ShapeDtypeStruct((_NW, _NP, _CPW), jnp.float32),
        mesh=_mesh(),
        scratch_types=[
            pltpu.VMEM((_NP, _CPW), jnp.float32),
            pltpu.VMEM((_RK, 128), jnp.float32),
            pltpu.VMEM((_RE, _CPW), jnp.float32),
            pltpu.VMEM((_RK,), jnp.int32),
            pltpu.VMEM((_RE,), jnp.int32),
            pltpu.VMEM_SHARED((16, _RE, _CPW), jnp.float32),
            pltpu.SemaphoreType.DMA,
        ],
        compiler_params=pltpu.CompilerParams(needs_layout_passes=False, use_tc_tiling_on_sc=False),
    )
    return kfn(y3, srcp, dstp, zer8)


def _mm(x, w):
    m, k = x.shape
    _, n = w.shape

    def body(x_ref, w_ref, o_ref):
        o_ref[...] = jnp.dot(x_ref[...], w_ref[...],
                             preferred_element_type=jnp.float32)

    return pl.pallas_call(
        body,
        grid=(m // _BR,),
        in_specs=[pl.BlockSpec((_BR, k), lambda i: (i, 0)),
                  pl.BlockSpec((k, n), lambda i: (0, 0))],
        out_specs=pl.BlockSpec((_BR, n), lambda i: (i, 0)),
        out_shape=jax.ShapeDtypeStruct((m, n), jnp.float32),
    )(x, w)


def _scale(xw, deg):
    def body(xw_ref, deg_ref, y_ref, dinv_ref):
        i = pl.program_id(0)
        row = i * _BR + lax.broadcasted_iota(jnp.int32, (_BR, 1), 0)
        d = deg_ref[...]
        dinv = jnp.where(row < _N, lax.rsqrt(d + 1.0), 0.0)
        y_ref[...] = xw_ref[...] * dinv
        dinv_ref[...] = dinv

    return pl.pallas_call(
        body,
        grid=(_NB,),
        in_specs=[pl.BlockSpec((_BR, _HID), lambda i: (i, 0)),
                  pl.BlockSpec((_BR, 1), lambda i: (i, 0))],
        out_specs=[pl.BlockSpec((_BR, _HID), lambda i: (i, 0)),
                   pl.BlockSpec((_BR, 1), lambda i: (i, 0))],
        out_shape=[jax.ShapeDtypeStruct((_NP, _HID), jnp.float32),
                   jax.ShapeDtypeStruct((_NP, 1), jnp.float32)],
    )(xw, deg)


def _layer_mid(z, y, dinv, b1, w2):
    def body(z_ref, y_ref, dinv_ref, b1_ref, w_ref, y2_ref):
        dv = dinv_ref[...]
        h = jnp.maximum(dv * (z_ref[...] + y_ref[...]) + b1_ref[...], 0.0)
        y2_ref[...] = dv * jnp.dot(h, w_ref[...],
                                   preferred_element_type=jnp.float32)

    return pl.pallas_call(
        body,
        grid=(_NB,),
        in_specs=[pl.BlockSpec((_BR, _HID), lambda i: (i, 0)),
                  pl.BlockSpec((_BR, _HID), lambda i: (i, 0)),
                  pl.BlockSpec((_BR, 1), lambda i: (i, 0)),
                  pl.BlockSpec((1, _HID), lambda i: (0, 0)),
                  pl.BlockSpec((_HID, _HID), lambda i: (0, 0))],
        out_specs=pl.BlockSpec((_BR, _HID), lambda i: (i, 0)),
        out_shape=jax.ShapeDtypeStruct((_NP, _HID), jnp.float32),
    )(z, y, dinv, b1, w2)


def _pool_head(z2, y2, dinv, b2, bpad, wl, bl):
    def body(z_ref, y_ref, dinv_ref, b2_ref, b_ref, wl_ref, bl_ref, o_ref,
             sum_acc, cnt_acc):
        i = pl.program_id(0)

        @pl.when(i == 0)
        def _():
            sum_acc[...] = jnp.zeros_like(sum_acc)
            cnt_acc[...] = jnp.zeros_like(cnt_acc)

        dv = dinv_ref[...]
        h2 = dv * (z_ref[...] + y_ref[...]) + b2_ref[...]
        bid = b_ref[...]
        gid = lax.broadcasted_iota(jnp.int32, (_BR, _G), 1)
        oh = (bid == gid).astype(jnp.float32)
        dn = (((0,), (0,)), ((), ()))
        sum_acc[...] += lax.dot_general(oh, h2, dn,
                                        preferred_element_type=jnp.float32)
        cnt_acc[...] += lax.dot_general(oh, jnp.ones((_BR, 128), jnp.float32),
                                        dn, preferred_element_type=jnp.float32)

        @pl.when(i == _NB - 1)
        def _():
            pooled = sum_acc[...] / jnp.maximum(cnt_acc[...][:, 0:1], 1.0)
            o_ref[...] = jnp.dot(pooled, wl_ref[...],
                                 preferred_element_type=jnp.float32) + bl_ref[...]

    return pl.pallas_call(
        body,
        grid=(_NB,),
        in_specs=[pl.BlockSpec((_BR, _HID), lambda i: (i, 0)),
                  pl.BlockSpec((_BR, _HID), lambda i: (i, 0)),
                  pl.BlockSpec((_BR, 1), lambda i: (i, 0)),
                  pl.BlockSpec((1, _HID), lambda i: (0, 0)),
                  pl.BlockSpec((_BR, 1), lambda i: (i, 0)),
                  pl.BlockSpec((_HID, _OUT_C), lambda i: (0, 0)),
                  pl.BlockSpec((1, _OUT_C), lambda i: (0, 0))],
        out_specs=pl.BlockSpec((_G, _OUT_C), lambda i: (0, 0)),
        out_shape=jax.ShapeDtypeStruct((_G, _OUT_C), jnp.float32),
        scratch_shapes=[pltpu.VMEM((_G, _HID), jnp.float32),
                        pltpu.VMEM((_G, 128), jnp.float32)],
    )(z2, y2, dinv, b2, bpad, wl, bl)


def _to_sc(y):
    return y.reshape(_NP, 2, 128).transpose(1, 0, 2)


def _from_sc(z3):
    return z3.reshape(2, 16, _NP, _CPW).transpose(2, 0, 1, 3).reshape(_NP, _HID)


def kernel(x, edge_index, batch, W1, b1, W2, b2, Wl, bl):
    src = edge_index[0].astype(jnp.int32)
    dst = edge_index[1].astype(jnp.int32)
    srcp = jnp.concatenate([src, jnp.zeros((_EPAD - _E,), jnp.int32)])
    dstp = jnp.concatenate([dst, jnp.full((_EPAD - _E,), _NP - 1, jnp.int32)])
    xpad = jnp.zeros((_NP, _IN_C), jnp.float32).at[:_N].set(x)
    bpad = jnp.full((_NP, 1), _G, jnp.int32).at[:_N, 0].set(batch.astype(jnp.int32))
    zer_np = jnp.zeros((_NP,), jnp.float32)
    zer_acc = jnp.zeros((_NP, _CPW), jnp.float32)

    deg = _sc_degree(dst, zer_np).reshape(_NP, 1)
    xw1 = _mm(xpad, W1)
    y1, dinv = _scale(xw1, deg)
    z1 = _from_sc(_sc_scatter(_to_sc(y1), srcp, dstp, zer_acc))
    y2 = _layer_mid(z1, y1, dinv, b1.reshape(1, _HID), W2)
    z2 = _from_sc(_sc_scatter(_to_sc(y2), srcp, dstp, zer_acc))
    return _pool_head(z2, y2, dinv, b2.reshape(1, _HID), bpad, Wl,
                      bl.reshape(1, _OUT_C))


# speedup vs baseline: 1.9489x; 1.2591x over previous
import jax
import jax.numpy as jnp
from jax import lax
from jax.experimental import pallas as pl
from jax.experimental.pallas import tpu as pltpu
from jax.experimental.pallas import tpu_sc as plsc

_N = 10000
_E = 320000
_IN_C = 128
_HID = 256
_OUT_C = 64
_G = 64

_NP = 10240
_NW = 32
_CPW = _HID // _NW

_DEPT = _E // _NW
_DK = 400
_DNCH = _DEPT // _DK
_RPW = _NP // _NW

_RK = 64
_RE = _RK * 16
_NR = 314
_EPAD = (_NR + 2) * _RE
_PAIRS = _RE // 2
_PU = 8

_BR = 256
_NB = _NP // _BR


def _mesh():
    return plsc.VectorSubcoreMesh(core_axis_name="c", subcore_axis_name="s")


def _wid():
    return lax.axis_index("c") * 16 + lax.axis_index("s")


def _sc_degree(dst, zer_np):
    def body(dst_hbm, zer_hbm, deg_hbm, part, hist, dacc, dbuf, didx):
        w = _wid()
        pltpu.sync_copy(zer_hbm, hist)
        ones16 = jnp.ones((16,), jnp.float32)

        def chunk(i, carry):
            base = pl.multiple_of(w * _DEPT + i * _DK, 8)
            pltpu.sync_copy(dst_hbm.at[pl.ds(base, _DK)], didx)
            for k in range(_DK // 16):
                d16 = didx[pl.ds(k * 16, 16)]
                plsc.addupdate_scatter(hist, [d16], ones16)
            return carry

        lax.fori_loop(0, _DNCH, chunk, 0)
        for r in range(_NW):
            pltpu.sync_copy(hist.at[pl.ds(r * _RPW, _RPW)],
                            part.at[w * _NW + r])
        plsc.subcore_barrier()

        pltpu.sync_copy(part.at[w], dacc)
        for t in range(1, _NW):
            pltpu.sync_copy(part.at[t * _NW + w], dbuf)
            for k in range(_RPW // 16):
                dacc[pl.ds(k * 16, 16)] += dbuf[pl.ds(k * 16, 16)]
        r0 = pl.multiple_of(w * _RPW, 8)
        pltpu.sync_copy(dacc, deg_hbm.at[pl.ds(r0, _RPW)])

    kfn = pl.kernel(
        body,
        out_type=jax.ShapeDtypeStruct((_NP,), jnp.float32),
        mesh=_mesh(),
        scratch_types=[
            pltpu.VMEM_SHARED((_NW * _NW, _RPW), jnp.float32),
            pltpu.VMEM((_NP,), jnp.float32),
            pltpu.VMEM((_RPW,), jnp.float32),
            pltpu.VMEM((_RPW,), jnp.float32),
            pltpu.VMEM((_DK,), jnp.int32),
        ],
        compiler_params=pltpu.CompilerParams(needs_layout_passes=False, use_tc_tiling_on_sc=False),
    )
    return kfn(dst, zer_np)


def _sc_scatter(y3, srcp, dstp, zer8):
    def body(y_hbm, src_hbm, dst_hbm, zer_hbm, z_hbm,
             acc, rows0, rows1, rbuf, sidx0, sidx1, didx0, didx1,
             sbuf0, sbuf1, sem_g, sem_d, sem_s, sem_t):
        c = lax.axis_index("c")
        s = lax.axis_index("s")
        w = c * 16 + s
        pltpu.sync_copy(zer_hbm, acc)
        iota = lax.iota(jnp.int32, 16)
        pat = jnp.right_shift(iota, 3)
        colofs = jnp.bitwise_and(iota, 7)

        rows_ = (rows0, rows1)
        sidx_ = (sidx0, sidx1)
        didx_ = (didx0, didx1)
        sbuf_ = (sbuf0, sbuf1)

        def sidx_base(r):
            return pl.multiple_of(r * _RE + s * _RK, 8)

        pltpu.sync_copy(src_hbm.at[pl.ds(sidx_base(0), _RK)], sidx0)
        pltpu.async_copy(y_hbm.at[c].at[plsc.Indices(sidx0)],
                         rows0, sem_g).wait()
        pltpu.sync_copy(dst_hbm.at[pl.ds(0, _RE)], didx0)

        def half(p, r):
            q = 1 - p
            pltpu.sync_copy(src_hbm.at[pl.ds(sidx_base(r + 1), _RK)],
                            sidx_[q])
            gcp = pltpu.async_copy(y_hbm.at[c].at[plsc.Indices(sidx_[q])],
                                   rows_[q], sem_g)
            e1 = pl.multiple_of((r + 1) * _RE, 8)
            dcp = pltpu.async_copy(dst_hbm.at[pl.ds(e1, _RE)],
                                   didx_[q], sem_d)
            cps = [
                pltpu.async_copy(rows_[p].at[:, pl.ds(d * _CPW, _CPW)],
                                 sbuf_[p].at[d, pl.ds(s * _RK, _RK)], sem_t)
                for d in range(16)
            ]
            for cp in cps:
                cp.wait()
            plsc.subcore_barrier()
            pltpu.sync_copy(sbuf_[p].at[s], rbuf)

            def pairs(pp, carry2):
                for u in range(_PU):
                    rpat = (pp * _PU + u) * 2 + pat
                    a16 = plsc.load_gather(didx_[p], [rpat])
                    v16 = plsc.load_gather(rbuf, [rpat, colofs])
                    plsc.addupdate_scatter(acc, [a16, colofs], v16)
                return carry2

            lax.fori_loop(0, _PAIRS // _PU, pairs, 0)
            gcp.wait()
            dcp.wait()
            plsc.subcore_barrier()

        def two_rounds(k, carry):
            half(0, 2 * k)
            half(1, 2 * k + 1)
            return carry

        lax.fori_loop(0, _NR // 2, two_rounds, 0)
        pltpu.sync_copy(acc, z_hbm.at[w])

    kfn = pl.kernel(
        body,
        out_type=jax.ShapeDtypeStruct((_NW, _NP, _CPW), jnp.float32),
        mesh=_mesh(),
        scratch_types=[
            pltpu.VMEM((_NP, _CPW), jnp.float32),
            pltpu.VMEM((_RK, 128), jnp.float32),
            pltpu.VMEM((_RK, 128), jnp.float32),
            pltpu.VMEM((_RE, _CPW), jnp.float32),
            pltpu.VMEM((_RK,), jnp.int32),
            pltpu.VMEM((_RK,), jnp.int32),
            pltpu.VMEM((_RE,), jnp.int32),
            pltpu.VMEM((_RE,), jnp.int32),
            pltpu.VMEM_SHARED((16, _RE, _CPW), jnp.float32),
            pltpu.VMEM_SHARED((16, _RE, _CPW), jnp.float32),
            pltpu.SemaphoreType.DMA,
            pltpu.SemaphoreType.DMA,
            pltpu.SemaphoreType.DMA,
            pltpu.SemaphoreType.DMA,
        ],
        compiler_params=pltpu.CompilerParams(needs_layout_passes=False, use_tc_tiling_on_sc=False),
    )
    return kfn(y3, srcp, dstp, zer8)


def _mm(x, w):
    m, k = x.shape
    _, n = w.shape

    def body(x_ref, w_ref, o_ref):
        o_ref[...] = jnp.dot(x_ref[...], w_ref[...],
                             preferred_element_type=jnp.float32)

    return pl.pallas_call(
        body,
        grid=(m // _BR,),
        in_specs=[pl.BlockSpec((_BR, k), lambda i: (i, 0)),
                  pl.BlockSpec((k, n), lambda i: (0, 0))],
        out_specs=pl.BlockSpec((_BR, n), lambda i: (i, 0)),
        out_shape=jax.ShapeDtypeStruct((m, n), jnp.float32),
    )(x, w)


def _scale(xw, deg):
    def body(xw_ref, deg_ref, y_ref, dinv_ref):
        i = pl.program_id(0)
        row = i * _BR + lax.broadcasted_iota(jnp.int32, (_BR, 1), 0)
        d = deg_ref[...]
        dinv = jnp.where(row < _N, lax.rsqrt(d + 1.0), 0.0)
        y_ref[...] = xw_ref[...] * dinv
        dinv_ref[...] = dinv

    return pl.pallas_call(
        body,
        grid=(_NB,),
        in_specs=[pl.BlockSpec((_BR, _HID), lambda i: (i, 0)),
                  pl.BlockSpec((_BR, 1), lambda i: (i, 0))],
        out_specs=[pl.BlockSpec((_BR, _HID), lambda i: (i, 0)),
                   pl.BlockSpec((_BR, 1), lambda i: (i, 0))],
        out_shape=[jax.ShapeDtypeStruct((_NP, _HID), jnp.float32),
                   jax.ShapeDtypeStruct((_NP, 1), jnp.float32)],
    )(xw, deg)


def _layer_mid(z, y, dinv, b1, w2):
    def body(z_ref, y_ref, dinv_ref, b1_ref, w_ref, y2_ref):
        dv = dinv_ref[...]
        h = jnp.maximum(dv * (z_ref[...] + y_ref[...]) + b1_ref[...], 0.0)
        y2_ref[...] = dv * jnp.dot(h, w_ref[...],
                                   preferred_element_type=jnp.float32)

    return pl.pallas_call(
        body,
        grid=(_NB,),
        in_specs=[pl.BlockSpec((_BR, _HID), lambda i: (i, 0)),
                  pl.BlockSpec((_BR, _HID), lambda i: (i, 0)),
                  pl.BlockSpec((_BR, 1), lambda i: (i, 0)),
                  pl.BlockSpec((1, _HID), lambda i: (0, 0)),
                  pl.BlockSpec((_HID, _HID), lambda i: (0, 0))],
        out_specs=pl.BlockSpec((_BR, _HID), lambda i: (i, 0)),
        out_shape=jax.ShapeDtypeStruct((_NP, _HID), jnp.float32),
    )(z, y, dinv, b1, w2)


def _pool_head(z2, y2, dinv, b2, bpad, wl, bl):
    def body(z_ref, y_ref, dinv_ref, b2_ref, b_ref, wl_ref, bl_ref, o_ref,
             sum_acc, cnt_acc):
        i = pl.program_id(0)

        @pl.when(i == 0)
        def _():
            sum_acc[...] = jnp.zeros_like(sum_acc)
            cnt_acc[...] = jnp.zeros_like(cnt_acc)

        dv = dinv_ref[...]
        h2 = dv * (z_ref[...] + y_ref[...]) + b2_ref[...]
        bid = b_ref[...]
        gid = lax.broadcasted_iota(jnp.int32, (_BR, _G), 1)
        oh = (bid == gid).astype(jnp.float32)
        dn = (((0,), (0,)), ((), ()))
        sum_acc[...] += lax.dot_general(oh, h2, dn,
                                        preferred_element_type=jnp.float32)
        cnt_acc[...] += lax.dot_general(oh, jnp.ones((_BR, 128), jnp.float32),
                                        dn, preferred_element_type=jnp.float32)

        @pl.when(i == _NB - 1)
        def _():
            pooled = sum_acc[...] / jnp.maximum(cnt_acc[...][:, 0:1], 1.0)
            o_ref[...] = jnp.dot(pooled, wl_ref[...],
                                 preferred_element_type=jnp.float32) + bl_ref[...]

    return pl.pallas_call(
        body,
        grid=(_NB,),
        in_specs=[pl.BlockSpec((_BR, _HID), lambda i: (i, 0)),
                  pl.BlockSpec((_BR, _HID), lambda i: (i, 0)),
                  pl.BlockSpec((_BR, 1), lambda i: (i, 0)),
                  pl.BlockSpec((1, _HID), lambda i: (0, 0)),
                  pl.BlockSpec((_BR, 1), lambda i: (i, 0)),
                  pl.BlockSpec((_HID, _OUT_C), lambda i: (0, 0)),
                  pl.BlockSpec((1, _OUT_C), lambda i: (0, 0))],
        out_specs=pl.BlockSpec((_G, _OUT_C), lambda i: (0, 0)),
        out_shape=jax.ShapeDtypeStruct((_G, _OUT_C), jnp.float32),
        scratch_shapes=[pltpu.VMEM((_G, _HID), jnp.float32),
                        pltpu.VMEM((_G, 128), jnp.float32)],
    )(z2, y2, dinv, b2, bpad, wl, bl)


def _to_sc(y):
    return y.reshape(_NP, 2, 128).transpose(1, 0, 2)


def _from_sc(z3):
    return z3.reshape(2, 16, _NP, _CPW).transpose(2, 0, 1, 3).reshape(_NP, _HID)


def kernel(x, edge_index, batch, W1, b1, W2, b2, Wl, bl):
    src = edge_index[0].astype(jnp.int32)
    dst = edge_index[1].astype(jnp.int32)
    srcp = jnp.concatenate([src, jnp.zeros((_EPAD - _E,), jnp.int32)])
    dstp = jnp.concatenate([dst, jnp.full((_EPAD - _E,), _NP - 1, jnp.int32)])
    xpad = jnp.zeros((_NP, _IN_C), jnp.float32).at[:_N].set(x)
    bpad = jnp.full((_NP, 1), _G, jnp.int32).at[:_N, 0].set(batch.astype(jnp.int32))
    zer_np = jnp.zeros((_NP,), jnp.float32)
    zer_acc = jnp.zeros((_NP, _CPW), jnp.float32)

    deg = _sc_degree(dst, zer_np).reshape(_NP, 1)
    xw1 = _mm(xpad, W1)
    y1, dinv = _scale(xw1, deg)
    z1 = _from_sc(_sc_scatter(_to_sc(y1), srcp, dstp, zer_acc))
    y2 = _layer_mid(z1, y1, dinv, b1.reshape(1, _HID), W2)
    z2 = _from_sc(_sc_scatter(_to_sc(y2), srcp, dstp, zer_acc))
    return _pool_head(z2, y2, dinv, b2.reshape(1, _HID), bpad, Wl,
                      bl.reshape(1, _OUT_C))


# speedup vs baseline: 2.0018x; 1.0271x over previous
import jax
import jax.numpy as jnp
from jax import lax
from jax.experimental import pallas as pl
from jax.experimental.pallas import tpu as pltpu
from jax.experimental.pallas import tpu_sc as plsc

_N = 10000
_E = 320000
_IN_C = 128
_HID = 256
_OUT_C = 64
_G = 64

_NP = 10240
_NW = 32
_CPW = _HID // _NW

_DEPT = _E // _NW
_DK = 400
_DNCH = _DEPT // _DK
_RPW = _NP // _NW

_RK = 72
_RE = _RK * 16
_NR = 278
_EPAD = (_NR + 2) * _RE
_PAIRS = _RE // 2
_PU = 8

_BR = 256
_NB = _NP // _BR


def _mesh():
    return plsc.VectorSubcoreMesh(core_axis_name="c", subcore_axis_name="s")


def _wid():
    return lax.axis_index("c") * 16 + lax.axis_index("s")


def _sc_degree(dst, zer_np):
    def body(dst_hbm, zer_hbm, deg_hbm, part, hist, dacc, dbuf, didx):
        w = _wid()
        pltpu.sync_copy(zer_hbm, hist)
        ones16 = jnp.ones((16,), jnp.float32)

        def chunk(i, carry):
            base = pl.multiple_of(w * _DEPT + i * _DK, 8)
            pltpu.sync_copy(dst_hbm.at[pl.ds(base, _DK)], didx)
            for k in range(_DK // 16):
                d16 = didx[pl.ds(k * 16, 16)]
                plsc.addupdate_scatter(hist, [d16], ones16)
            return carry

        lax.fori_loop(0, _DNCH, chunk, 0)
        for r in range(_NW):
            pltpu.sync_copy(hist.at[pl.ds(r * _RPW, _RPW)],
                            part.at[w * _NW + r])
        plsc.subcore_barrier()

        pltpu.sync_copy(part.at[w], dacc)
        for t in range(1, _NW):
            pltpu.sync_copy(part.at[t * _NW + w], dbuf)
            for k in range(_RPW // 16):
                dacc[pl.ds(k * 16, 16)] += dbuf[pl.ds(k * 16, 16)]
        r0 = pl.multiple_of(w * _RPW, 8)
        pltpu.sync_copy(dacc, deg_hbm.at[pl.ds(r0, _RPW)])

    kfn = pl.kernel(
        body,
        out_type=jax.ShapeDtypeStruct((_NP,), jnp.float32),
        mesh=_mesh(),
        scratch_types=[
            pltpu.VMEM_SHARED((_NW * _NW, _RPW), jnp.float32),
            pltpu.VMEM((_NP,), jnp.float32),
            pltpu.VMEM((_RPW,), jnp.float32),
            pltpu.VMEM((_RPW,), jnp.float32),
            pltpu.VMEM((_DK,), jnp.int32),
        ],
        compiler_params=pltpu.CompilerParams(needs_layout_passes=False, use_tc_tiling_on_sc=False),
    )
    return kfn(dst, zer_np)


def _sc_scatter(y3, srcp, dstp, zer8):
    def body(y_hbm, src_hbm, dst_hbm, zer_hbm, z_hbm,
             acc, rows0, rows1, rbuf, sidx0, sidx1, didx0, didx1,
             sbuf0, sbuf1, sem_g, sem_d, sem_s, sem_t):
        c = lax.axis_index("c")
        s = lax.axis_index("s")
        w = c * 16 + s
        pltpu.sync_copy(zer_hbm, acc)
        iota = lax.iota(jnp.int32, 16)
        pat = jnp.right_shift(iota, 3)
        colofs = jnp.bitwise_and(iota, 7)

        rows_ = (rows0, rows1)
        sidx_ = (sidx0, sidx1)
        didx_ = (didx0, didx1)
        sbuf_ = (sbuf0, sbuf1)

        def sidx_base(r):
            return pl.multiple_of(r * _RE + s * _RK, 8)

        pltpu.sync_copy(src_hbm.at[pl.ds(sidx_base(0), _RK)], sidx0)
        pltpu.async_copy(y_hbm.at[c].at[plsc.Indices(sidx0)],
                         rows0, sem_g).wait()
        pltpu.sync_copy(dst_hbm.at[pl.ds(0, _RE)], didx0)

        def half(p, r):
            q = 1 - p
            pltpu.sync_copy(src_hbm.at[pl.ds(sidx_base(r + 1), _RK)],
                            sidx_[q])
            gcp = pltpu.async_copy(y_hbm.at[c].at[plsc.Indices(sidx_[q])],
                                   rows_[q], sem_g)
            e1 = pl.multiple_of((r + 1) * _RE, 8)
            dcp = pltpu.async_copy(dst_hbm.at[pl.ds(e1, _RE)],
                                   didx_[q], sem_d)
            cps = [
                pltpu.async_copy(rows_[p].at[:, pl.ds(d * _CPW, _CPW)],
                                 sbuf_[p].at[d, pl.ds(s * _RK, _RK)], sem_t)
                for d in range(16)
            ]
            for cp in cps:
                cp.wait()
            plsc.subcore_barrier()
            pltpu.sync_copy(sbuf_[p].at[s], rbuf)

            def pairs(pp, carry2):
                for u in range(_PU):
                    rpat = (pp * _PU + u) * 2 + pat
                    a16 = plsc.load_gather(didx_[p], [rpat])
                    v16 = plsc.load_gather(rbuf, [rpat, colofs])
                    plsc.addupdate_scatter(acc, [a16, colofs], v16)
                return carry2

            lax.fori_loop(0, _PAIRS // _PU, pairs, 0)
            gcp.wait()
            dcp.wait()
            plsc.subcore_barrier()

        def two_rounds(k, carry):
            half(0, 2 * k)
            half(1, 2 * k + 1)
            return carry

        lax.fori_loop(0, _NR // 2, two_rounds, 0)
        pltpu.sync_copy(acc, z_hbm.at[w])

    kfn = pl.kernel(
        body,
        out_type=jax.ShapeDtypeStruct((_NW, _NP, _CPW), jnp.float32),
        mesh=_mesh(),
        scratch_types=[
            pltpu.VMEM((_NP, _CPW), jnp.float32),
            pltpu.VMEM((_RK, 128), jnp.float32),
            pltpu.VMEM((_RK, 128), jnp.float32),
            pltpu.VMEM((_RE, _CPW), jnp.float32),
            pltpu.VMEM((_RK,), jnp.int32),
            pltpu.VMEM((_RK,), jnp.int32),
            pltpu.VMEM((_RE,), jnp.int32),
            pltpu.VMEM((_RE,), jnp.int32),
            pltpu.VMEM_SHARED((16, _RE, _CPW), jnp.float32),
            pltpu.VMEM_SHARED((16, _RE, _CPW), jnp.float32),
            pltpu.SemaphoreType.DMA,
            pltpu.SemaphoreType.DMA,
            pltpu.SemaphoreType.DMA,
            pltpu.SemaphoreType.DMA,
        ],
        compiler_params=pltpu.CompilerParams(needs_layout_passes=False, use_tc_tiling_on_sc=False),
    )
    return kfn(y3, srcp, dstp, zer8)


def _mm(x, w):
    m, k = x.shape
    _, n = w.shape

    def body(x_ref, w_ref, o_ref):
        o_ref[...] = jnp.dot(x_ref[...], w_ref[...],
                             preferred_element_type=jnp.float32)

    return pl.pallas_call(
        body,
        grid=(m // _BR,),
        in_specs=[pl.BlockSpec((_BR, k), lambda i: (i, 0)),
                  pl.BlockSpec((k, n), lambda i: (0, 0))],
        out_specs=pl.BlockSpec((_BR, n), lambda i: (i, 0)),
        out_shape=jax.ShapeDtypeStruct((m, n), jnp.float32),
    )(x, w)


def _scale(xw, deg):
    def body(xw_ref, deg_ref, y_ref, dinv_ref):
        i = pl.program_id(0)
        row = i * _BR + lax.broadcasted_iota(jnp.int32, (_BR, 1), 0)
        d = deg_ref[...]
        dinv = jnp.where(row < _N, lax.rsqrt(d + 1.0), 0.0)
        y_ref[...] = xw_ref[...] * dinv
        dinv_ref[...] = dinv

    return pl.pallas_call(
        body,
        grid=(_NB,),
        in_specs=[pl.BlockSpec((_BR, _HID), lambda i: (i, 0)),
                  pl.BlockSpec((_BR, 1), lambda i: (i, 0))],
        out_specs=[pl.BlockSpec((_BR, _HID), lambda i: (i, 0)),
                   pl.BlockSpec((_BR, 1), lambda i: (i, 0))],
        out_shape=[jax.ShapeDtypeStruct((_NP, _HID), jnp.float32),
                   jax.ShapeDtypeStruct((_NP, 1), jnp.float32)],
    )(xw, deg)


def _layer_mid(z, y, dinv, b1, w2):
    def body(z_ref, y_ref, dinv_ref, b1_ref, w_ref, y2_ref):
        dv = dinv_ref[...]
        h = jnp.maximum(dv * (z_ref[...] + y_ref[...]) + b1_ref[...], 0.0)
        y2_ref[...] = dv * jnp.dot(h, w_ref[...],
                                   preferred_element_type=jnp.float32)

    return pl.pallas_call(
        body,
        grid=(_NB,),
        in_specs=[pl.BlockSpec((_BR, _HID), lambda i: (i, 0)),
                  pl.BlockSpec((_BR, _HID), lambda i: (i, 0)),
                  pl.BlockSpec((_BR, 1), lambda i: (i, 0)),
                  pl.BlockSpec((1, _HID), lambda i: (0, 0)),
                  pl.BlockSpec((_HID, _HID), lambda i: (0, 0))],
        out_specs=pl.BlockSpec((_BR, _HID), lambda i: (i, 0)),
        out_shape=jax.ShapeDtypeStruct((_NP, _HID), jnp.float32),
    )(z, y, dinv, b1, w2)


def _pool_head(z2, y2, dinv, b2, bpad, wl, bl):
    def body(z_ref, y_ref, dinv_ref, b2_ref, b_ref, wl_ref, bl_ref, o_ref,
             sum_acc, cnt_acc):
        i = pl.program_id(0)

        @pl.when(i == 0)
        def _():
            sum_acc[...] = jnp.zeros_like(sum_acc)
            cnt_acc[...] = jnp.zeros_like(cnt_acc)

        dv = dinv_ref[...]
        h2 = dv * (z_ref[...] + y_ref[...]) + b2_ref[...]
        bid = b_ref[...]
        gid = lax.broadcasted_iota(jnp.int32, (_BR, _G), 1)
        oh = (bid == gid).astype(jnp.float32)
        dn = (((0,), (0,)), ((), ()))
        sum_acc[...] += lax.dot_general(oh, h2, dn,
                                        preferred_element_type=jnp.float32)
        cnt_acc[...] += lax.dot_general(oh, jnp.ones((_BR, 128), jnp.float32),
                                        dn, preferred_element_type=jnp.float32)

        @pl.when(i == _NB - 1)
        def _():
            pooled = sum_acc[...] / jnp.maximum(cnt_acc[...][:, 0:1], 1.0)
            o_ref[...] = jnp.dot(pooled, wl_ref[...],
                                 preferred_element_type=jnp.float32) + bl_ref[...]

    return pl.pallas_call(
        body,
        grid=(_NB,),
        in_specs=[pl.BlockSpec((_BR, _HID), lambda i: (i, 0)),
                  pl.BlockSpec((_BR, _HID), lambda i: (i, 0)),
                  pl.BlockSpec((_BR, 1), lambda i: (i, 0)),
                  pl.BlockSpec((1, _HID), lambda i: (0, 0)),
                  pl.BlockSpec((_BR, 1), lambda i: (i, 0)),
                  pl.BlockSpec((_HID, _OUT_C), lambda i: (0, 0)),
                  pl.BlockSpec((1, _OUT_C), lambda i: (0, 0))],
        out_specs=pl.BlockSpec((_G, _OUT_C), lambda i: (0, 0)),
        out_shape=jax.ShapeDtypeStruct((_G, _OUT_C), jnp.float32),
        scratch_shapes=[pltpu.VMEM((_G, _HID), jnp.float32),
                        pltpu.VMEM((_G, 128), jnp.float32)],
    )(z2, y2, dinv, b2, bpad, wl, bl)


def _to_sc(y):
    return y.reshape(_NP, 2, 128).transpose(1, 0, 2)


def _from_sc(z3):
    return z3.reshape(2, 16, _NP, _CPW).transpose(2, 0, 1, 3).reshape(_NP, _HID)


def kernel(x, edge_index, batch, W1, b1, W2, b2, Wl, bl):
    src = edge_index[0].astype(jnp.int32)
    dst = edge_index[1].astype(jnp.int32)
    srcp = jnp.concatenate([src, jnp.zeros((_EPAD - _E,), jnp.int32)])
    dstp = jnp.concatenate([dst, jnp.full((_EPAD - _E,), _NP - 1, jnp.int32)])
    xpad = jnp.zeros((_NP, _IN_C), jnp.float32).at[:_N].set(x)
    bpad = jnp.full((_NP, 1), _G, jnp.int32).at[:_N, 0].set(batch.astype(jnp.int32))
    zer_np = jnp.zeros((_NP,), jnp.float32)
    zer_acc = jnp.zeros((_NP, _CPW), jnp.float32)

    deg = _sc_degree(dst, zer_np).reshape(_NP, 1)
    xw1 = _mm(xpad, W1)
    y1, dinv = _scale(xw1, deg)
    z1 = _from_sc(_sc_scatter(_to_sc(y1), srcp, dstp, zer_acc))
    y2 = _layer_mid(z1, y1, dinv, b1.reshape(1, _HID), W2)
    z2 = _from_sc(_sc_scatter(_to_sc(y2), srcp, dstp, zer_acc))
    return _pool_head(z2, y2, dinv, b2.reshape(1, _HID), bpad, Wl,
                      bl.reshape(1, _OUT_C))
